# baseline (speedup 1.0000x reference)
import jax
import jax.numpy as jnp
from jax import lax
from jax.experimental import pallas as pl
from jax.experimental.pallas import tpu as pltpu
from jax.experimental.pallas import tpu_sc as plsc

EMB = 32
NUM_ROLES = 12
NC, NS = 2, 16
NW = NC * NS


def _gather_body(idx_hbm, win_hbm, wout_hbm, out_hbm, idx_v, rows_v, sem):
    B = idx_hbm.shape[0] // NUM_ROLES
    n = B // NW
    wid = lax.axis_index("s") * NC + lax.axis_index("c")
    base = wid * n
    for r in range(NUM_ROLES):
        tab = win_hbm if r < 6 else wout_hbm
        off = r * B + base
        pltpu.sync_copy(idx_hbm.at[pl.ds(off, n)], idx_v)
        pltpu.async_copy(tab.at[idx_v], rows_v, sem).wait()
        pltpu.sync_copy(rows_v, out_hbm.at[pl.ds(off, n)])


def _dense_body(g_ref, out_ref):
    @pl.when(pl.program_id(0) == 0)
    def _():
        out_ref[0, 0] = jnp.float32(0.0)
        out_ref[0, 1] = jnp.float32(0.0)

    eps = 1e-12
    cm = (g_ref[0] + g_ref[1] + g_ref[2] + g_ref[3] + g_ref[4] + g_ref[5]) * (1.0 / 6.0)
    t = g_ref[6]
    tt = jnp.sum(t * t, axis=1)
    cc = jnp.sum(cm * cm, axis=1)
    tc = jnp.sum(t * cm, axis=1)
    rt = lax.rsqrt(jnp.maximum(tt, eps))
    rc = lax.rsqrt(jnp.maximum(cc, eps))
    pos = jnp.sum(jax.nn.sigmoid(tc * rt * rc))
    neg = jnp.float32(0.0)
    for j in range(5):
        nrow = g_ref[7 + j]
        nn = jnp.sum(nrow * nrow, axis=1)
        tn = jnp.sum(t * nrow, axis=1)
        rn = lax.rsqrt(jnp.maximum(nn, eps))
        neg = neg + jnp.sum(jax.nn.sigmoid(-(tn * rt * rn)))
    out_ref[0, 0] += pos
    out_ref[0, 1] += neg


def kernel(contexts, target, negatives, W_in, W_out):
    B = contexts.shape[0]
    idx = jnp.concatenate([
        contexts.T.reshape(-1),
        target.T.reshape(-1),
        negatives.T.reshape(-1),
    ]).astype(jnp.int32)

    n = B // NW
    mesh = plsc.VectorSubcoreMesh(core_axis_name="c", subcore_axis_name="s")
    gathered = pl.kernel(
        _gather_body,
        out_type=jax.ShapeDtypeStruct((NUM_ROLES * B, EMB), jnp.float32),
        mesh=mesh,
        scratch_types=[
            pltpu.VMEM((n,), jnp.int32),
            pltpu.VMEM((n, EMB), jnp.float32),
            pltpu.SemaphoreType.DMA,
        ],
        compiler_params=pltpu.CompilerParams(use_tc_tiling_on_sc=False),
    )(idx, W_in, W_out)

    g3 = gathered.reshape(NUM_ROLES, B, EMB)
    R = 2048
    partial = pl.pallas_call(
        _dense_body,
        grid=(B // R,),
        in_specs=[pl.BlockSpec((NUM_ROLES, R, EMB), lambda i: (0, i, 0))],
        out_specs=pl.BlockSpec((1, 2), lambda i: (0, 0), memory_space=pltpu.SMEM),
        out_shape=jax.ShapeDtypeStruct((1, 2), jnp.float32),
    )(g3)
    return partial[0, 0] / B + partial[0, 1] / (5 * B)
